# baseline (speedup 1.0000x reference)
import functools

import jax
import jax.numpy as jnp
from jax.experimental import pallas as pl
from jax.experimental.pallas import tpu as pltpu

_PRECISION = jax.lax.Precision.HIGHEST


def _cheb_kernel(coeffs_ref, l_ref, x_ref, out_ref,
                 t_scr, sum_scr, *, n_passes, bm):
    s = pl.program_id(0)
    i = pl.program_id(1)

    @pl.when(jnp.logical_and(s == 0, i == 0))
    def _():
        t_scr[0] = x_ref[...]

    src = s % 2
    acc = jax.lax.dot_general(
        l_ref[...], t_scr[src], (((1,), (0,)), ((), ())),
        preferred_element_type=jnp.float32, precision=_PRECISION)

    rows = pl.ds(i * bm, bm)

    @pl.when(s == 0)
    def _():
        t_scr[1, rows, :] = acc
        sum_scr[rows, :] = coeffs_ref[0] * x_ref[rows, :] + coeffs_ref[1] * acc

    @pl.when(s > 0)
    def _():
        dst = 1 - src
        new = 2.0 * acc - t_scr[dst, rows, :]
        t_scr[dst, rows, :] = new
        sum_scr[rows, :] += coeffs_ref[s + 1] * new

    @pl.when(s == n_passes - 1)
    def _():
        out_ref[...] = sum_scr[rows, :]


def kernel(L_rescaled, X, coeffs):
    n, d = X.shape
    m = coeffs.shape[0]
    n_passes = m - 1
    bm = 200 if n % 200 == 0 else n
    ni = n // bm
    return pl.pallas_call(
        functools.partial(_cheb_kernel, n_passes=n_passes, bm=bm),
        grid=(n_passes, ni),
        in_specs=[
            pl.BlockSpec(memory_space=pltpu.SMEM),
            pl.BlockSpec((bm, n), lambda s, i: (i, 0)),
            pl.BlockSpec((n, d), lambda s, i: (0, 0)),
        ],
        out_specs=pl.BlockSpec((bm, d), lambda s, i: (i, 0)),
        out_shape=jax.ShapeDtypeStruct((n, d), jnp.float32),
        scratch_shapes=[
            pltpu.VMEM((2, n, d), jnp.float32),
            pltpu.VMEM((n, d), jnp.float32),
        ],
        compiler_params=pltpu.CompilerParams(
            dimension_semantics=("arbitrary", "arbitrary")),
    )(coeffs, L_rescaled, X)


# speedup vs baseline: 2.7799x; 2.7799x over previous
import functools

import jax
import jax.numpy as jnp
from jax.experimental import pallas as pl
from jax.experimental.pallas import tpu as pltpu

_PRECISION = jax.lax.Precision.DEFAULT


def _cheb_kernel(coeffs_ref, l_ref, x_ref, out_ref,
                 t_scr, sum_scr, *, n_passes, bm):
    s = pl.program_id(0)
    i = pl.program_id(1)

    @pl.when(jnp.logical_and(s == 0, i == 0))
    def _():
        t_scr[0] = x_ref[...]

    src = s % 2
    acc = jax.lax.dot_general(
        l_ref[...], t_scr[src], (((1,), (0,)), ((), ())),
        preferred_element_type=jnp.float32, precision=_PRECISION)

    rows = pl.ds(i * bm, bm)

    @pl.when(s == 0)
    def _():
        t_scr[1, rows, :] = acc
        sum_scr[rows, :] = coeffs_ref[0] * x_ref[rows, :] + coeffs_ref[1] * acc

    @pl.when(s > 0)
    def _():
        dst = 1 - src
        new = 2.0 * acc - t_scr[dst, rows, :]
        t_scr[dst, rows, :] = new
        sum_scr[rows, :] += coeffs_ref[s + 1] * new

    @pl.when(s == n_passes - 1)
    def _():
        out_ref[...] = sum_scr[rows, :]


def kernel(L_rescaled, X, coeffs):
    n, d = X.shape
    m = coeffs.shape[0]
    n_passes = m - 1
    bm = 200 if n % 200 == 0 else n
    ni = n // bm
    return pl.pallas_call(
        functools.partial(_cheb_kernel, n_passes=n_passes, bm=bm),
        grid=(n_passes, ni),
        in_specs=[
            pl.BlockSpec(memory_space=pltpu.SMEM),
            pl.BlockSpec((bm, n), lambda s, i: (i, 0)),
            pl.BlockSpec((n, d), lambda s, i: (0, 0)),
        ],
        out_specs=pl.BlockSpec((bm, d), lambda s, i: (i, 0)),
        out_shape=jax.ShapeDtypeStruct((n, d), jnp.float32),
        scratch_shapes=[
            pltpu.VMEM((2, n, d), jnp.float32),
            pltpu.VMEM((n, d), jnp.float32),
        ],
        compiler_params=pltpu.CompilerParams(
            dimension_semantics=("arbitrary", "arbitrary")),
    )(coeffs, L_rescaled, X)


# speedup vs baseline: 3.2505x; 1.1693x over previous
import functools

import jax
import jax.numpy as jnp
from jax.experimental import pallas as pl
from jax.experimental.pallas import tpu as pltpu

_DOT = functools.partial(
    jax.lax.dot_general,
    dimension_numbers=(((1,), (0,)), ((), ())),
    preferred_element_type=jnp.float32,
    precision=jax.lax.Precision.DEFAULT)


def _stage_kernel(l_ref, x_ref, lbf_ref, t1_ref):
    lb = l_ref[...]
    lbf_ref[...] = lb.astype(jnp.bfloat16)
    t1_ref[...] = _DOT(lb, x_ref[...])


def _rec_kernel(coeffs_ref, lbf_ref, x_ref, t1_ref, out_ref,
                t_scr, sum_scr, *, n_rec, bm):
    s = pl.program_id(0)
    i = pl.program_id(1)

    @pl.when(jnp.logical_and(s == 0, i == 0))
    def _():
        t_scr[0] = x_ref[...].astype(jnp.bfloat16)
        t_scr[1] = t1_ref[...].astype(jnp.bfloat16)
        sum_scr[...] = coeffs_ref[0] * x_ref[...] + coeffs_ref[1] * t1_ref[...]

    src = 1 - s % 2
    dst = s % 2
    acc = _DOT(lbf_ref[...], t_scr[src])
    rows = pl.ds(i * bm, bm)
    new = 2.0 * acc - t_scr[dst, rows, :].astype(jnp.float32)
    t_scr[dst, rows, :] = new.astype(jnp.bfloat16)
    sum_scr[rows, :] += coeffs_ref[s + 2] * new

    @pl.when(s == n_rec - 1)
    def _():
        out_ref[...] = sum_scr[rows, :]


def kernel(L_rescaled, X, coeffs):
    n, d = X.shape
    m = coeffs.shape[0]
    bm = 200 if n % 200 == 0 else n
    ni = n // bm
    lbf, t1 = pl.pallas_call(
        _stage_kernel,
        grid=(ni,),
        in_specs=[
            pl.BlockSpec((bm, n), lambda i: (i, 0)),
            pl.BlockSpec((n, d), lambda i: (0, 0)),
        ],
        out_specs=[
            pl.BlockSpec((bm, n), lambda i: (i, 0)),
            pl.BlockSpec((bm, d), lambda i: (i, 0)),
        ],
        out_shape=[
            jax.ShapeDtypeStruct((n, n), jnp.bfloat16),
            jax.ShapeDtypeStruct((n, d), jnp.float32),
        ],
        compiler_params=pltpu.CompilerParams(
            dimension_semantics=("arbitrary",)),
    )(L_rescaled, X)

    n_rec = m - 2
    return pl.pallas_call(
        functools.partial(_rec_kernel, n_rec=n_rec, bm=bm),
        grid=(n_rec, ni),
        in_specs=[
            pl.BlockSpec(memory_space=pltpu.SMEM),
            pl.BlockSpec((bm, n), lambda s, i: (i, 0)),
            pl.BlockSpec((n, d), lambda s, i: (0, 0)),
            pl.BlockSpec((n, d), lambda s, i: (0, 0)),
        ],
        out_specs=pl.BlockSpec((bm, d), lambda s, i: (i, 0)),
        out_shape=jax.ShapeDtypeStruct((n, d), jnp.float32),
        scratch_shapes=[
            pltpu.VMEM((2, n, d), jnp.bfloat16),
            pltpu.VMEM((n, d), jnp.float32),
        ],
        compiler_params=pltpu.CompilerParams(
            dimension_semantics=("arbitrary", "arbitrary")),
    )(coeffs, lbf, X, t1)


# speedup vs baseline: 3.7765x; 1.1618x over previous
import functools

import jax
import jax.numpy as jnp
from jax.experimental import pallas as pl
from jax.experimental.pallas import tpu as pltpu

_DOT = functools.partial(
    jax.lax.dot_general,
    dimension_numbers=(((1,), (0,)), ((), ())),
    preferred_element_type=jnp.float32,
    precision=jax.lax.Precision.DEFAULT)


def _stage_kernel(l_ref, x_ref, lbf_ref, t1_ref):
    lb = l_ref[...]
    lbf_ref[...] = lb.astype(jnp.bfloat16)
    t1_ref[...] = _DOT(lb, x_ref[...])


def _rec_kernel(coeffs_ref, lbf_ref, x_ref, t1_ref, out_ref,
                t_scr, sum_scr, *, n_rec, bm):
    s = pl.program_id(0)
    i = pl.program_id(1)

    @pl.when(jnp.logical_and(s == 0, i == 0))
    def _():
        t_scr[0] = x_ref[...].astype(jnp.bfloat16)
        t_scr[1] = t1_ref[...].astype(jnp.bfloat16)
        sum_scr[...] = coeffs_ref[0] * x_ref[...] + coeffs_ref[1] * t1_ref[...]

    src = 1 - s % 2
    dst = s % 2
    acc = _DOT(lbf_ref[...], t_scr[src])
    rows = pl.ds(i * bm, bm)
    new = 2.0 * acc - t_scr[dst, rows, :].astype(jnp.float32)
    t_scr[dst, rows, :] = new.astype(jnp.bfloat16)
    sum_scr[rows, :] += coeffs_ref[s + 2] * new

    @pl.when(s == n_rec - 1)
    def _():
        out_ref[...] = sum_scr[rows, :]


def kernel(L_rescaled, X, coeffs):
    n, d = X.shape
    m = coeffs.shape[0]
    bm_a = 200 if n % 200 == 0 else n
    ni_a = n // bm_a
    lbf, t1 = pl.pallas_call(
        _stage_kernel,
        grid=(ni_a,),
        in_specs=[
            pl.BlockSpec((bm_a, n), lambda i: (i, 0)),
            pl.BlockSpec((n, d), lambda i: (0, 0)),
        ],
        out_specs=[
            pl.BlockSpec((bm_a, n), lambda i: (i, 0)),
            pl.BlockSpec((bm_a, d), lambda i: (i, 0)),
        ],
        out_shape=[
            jax.ShapeDtypeStruct((n, n), jnp.bfloat16),
            jax.ShapeDtypeStruct((n, d), jnp.float32),
        ],
        compiler_params=pltpu.CompilerParams(
            dimension_semantics=("arbitrary",)),
    )(L_rescaled, X)

    n_rec = m - 2
    bm = 400 if n % 400 == 0 else n
    ni = n // bm
    return pl.pallas_call(
        functools.partial(_rec_kernel, n_rec=n_rec, bm=bm),
        grid=(n_rec, ni),
        in_specs=[
            pl.BlockSpec(memory_space=pltpu.SMEM),
            pl.BlockSpec((bm, n), lambda s, i: (i, 0)),
            pl.BlockSpec((n, d), lambda s, i: (0, 0)),
            pl.BlockSpec((n, d), lambda s, i: (0, 0)),
        ],
        out_specs=pl.BlockSpec((bm, d), lambda s, i: (i, 0)),
        out_shape=jax.ShapeDtypeStruct((n, d), jnp.float32),
        scratch_shapes=[
            pltpu.VMEM((2, n, d), jnp.bfloat16),
            pltpu.VMEM((n, d), jnp.float32),
        ],
        compiler_params=pltpu.CompilerParams(
            dimension_semantics=("arbitrary", "arbitrary")),
    )(coeffs, lbf, X, t1)


# speedup vs baseline: 4.0947x; 1.0843x over previous
import functools

import jax
import jax.numpy as jnp
from jax.experimental import pallas as pl
from jax.experimental.pallas import tpu as pltpu

_DOT = functools.partial(
    jax.lax.dot_general,
    dimension_numbers=(((1,), (0,)), ((), ())),
    preferred_element_type=jnp.float32,
    precision=jax.lax.Precision.DEFAULT)


def _stage_kernel(l_ref, x_ref, lbf_ref, t1_ref):
    lb = l_ref[...].astype(jnp.bfloat16)
    lbf_ref[...] = lb
    t1_ref[...] = _DOT(lb, x_ref[...]).astype(jnp.bfloat16)


def _rec_kernel(coeffs_ref, lbf_ref, x_hbm, t1_hbm, out_ref,
                t_scr, sum_scr, sem, *, n_rec, bm):
    s = pl.program_id(0)
    i = pl.program_id(1)

    @pl.when(jnp.logical_and(s == 0, i == 0))
    def _():
        cp0 = pltpu.make_async_copy(x_hbm, t_scr.at[0], sem.at[0])
        cp1 = pltpu.make_async_copy(t1_hbm, t_scr.at[1], sem.at[1])
        cp0.start()
        cp1.start()
        cp0.wait()
        cp1.wait()

    src = 1 - s % 2
    dst = s % 2
    acc = _DOT(lbf_ref[...], t_scr[src])
    rows = pl.ds(i * bm, bm)
    t_prev = t_scr[dst, rows, :].astype(jnp.float32)
    new = 2.0 * acc - t_prev
    t_scr[dst, rows, :] = new.astype(jnp.bfloat16)

    @pl.when(s == 0)
    def _():
        sum_scr[rows, :] = (coeffs_ref[0] * t_prev
                            + coeffs_ref[1] * t_scr[src, rows, :].astype(jnp.float32)
                            + coeffs_ref[2] * new)

    @pl.when(s > 0)
    def _():
        sum_scr[rows, :] += coeffs_ref[s + 2] * new

    @pl.when(s == n_rec - 1)
    def _():
        out_ref[...] = sum_scr[rows, :]


def kernel(L_rescaled, X, coeffs):
    n, d = X.shape
    m = coeffs.shape[0]
    x_bf = X.astype(jnp.bfloat16)

    bm_a = 400 if n % 400 == 0 else n
    ni_a = n // bm_a
    lbf, t1 = pl.pallas_call(
        _stage_kernel,
        grid=(ni_a,),
        in_specs=[
            pl.BlockSpec((bm_a, n), lambda i: (i, 0)),
            pl.BlockSpec((n, d), lambda i: (0, 0)),
        ],
        out_specs=[
            pl.BlockSpec((bm_a, n), lambda i: (i, 0)),
            pl.BlockSpec((bm_a, d), lambda i: (i, 0)),
        ],
        out_shape=[
            jax.ShapeDtypeStruct((n, n), jnp.bfloat16),
            jax.ShapeDtypeStruct((n, d), jnp.bfloat16),
        ],
        compiler_params=pltpu.CompilerParams(
            dimension_semantics=("arbitrary",)),
    )(L_rescaled, x_bf)

    n_rec = m - 2
    bm = 1000 if n % 1000 == 0 else n
    ni = n // bm
    return pl.pallas_call(
        functools.partial(_rec_kernel, n_rec=n_rec, bm=bm),
        grid=(n_rec, ni),
        in_specs=[
            pl.BlockSpec(memory_space=pltpu.SMEM),
            pl.BlockSpec((bm, n), lambda s, i: (i, 0)),
            pl.BlockSpec(memory_space=pl.ANY),
            pl.BlockSpec(memory_space=pl.ANY),
        ],
        out_specs=pl.BlockSpec((bm, d), lambda s, i: (i, 0)),
        out_shape=jax.ShapeDtypeStruct((n, d), jnp.float32),
        scratch_shapes=[
            pltpu.VMEM((2, n, d), jnp.bfloat16),
            pltpu.VMEM((n, d), jnp.float32),
            pltpu.SemaphoreType.DMA((2,)),
        ],
        compiler_params=pltpu.CompilerParams(
            dimension_semantics=("arbitrary", "arbitrary")),
    )(coeffs, lbf, x_bf, t1)
